# Initial kernel scaffold; baseline (speedup 1.0000x reference)
import jax
import jax.numpy as jnp
from jax.experimental import pallas as pl


def kernel(x_vector, W1, b1, W2, b2, W3, b3, W4, b4, bn_gamma, bn_beta, bn_mean, bn_var, topic_course):
    raise NotImplementedError("write your pallas kernel here")


# speedup vs baseline: 8.0845x; 8.0845x over previous
import functools

import jax
import jax.numpy as jnp
from jax.experimental import pallas as pl

_B = 16384
_F = 128
_H = 256
_C = 91
_K = 45
_FILL = 0.05
_BM = 1024


def _fused_kernel(x_ref, w1_ref, b1_ref, w2_ref, b2_ref, w3_ref, b3_ref,
                  w4_ref, b4_ref, tct_ref, logits_ref, rt_ref):
    x = x_ref[...]
    h = jnp.maximum(jnp.dot(x, w1_ref[...], preferred_element_type=jnp.float32)
                    + b1_ref[...], 0.0)
    h = jnp.maximum(jnp.dot(h, w2_ref[...], preferred_element_type=jnp.float32)
                    + b2_ref[...], 0.0)
    h = jnp.maximum(jnp.dot(h, w3_ref[...], preferred_element_type=jnp.float32)
                    + b3_ref[...], 0.0)
    logits = jnp.dot(h, w4_ref[...], preferred_element_type=jnp.float32) + b4_ref[...]
    logits_ref[...] = logits

    col = jax.lax.broadcasted_iota(jnp.int32, (_BM, _C), 1)
    rank = jnp.zeros((_BM, _C), dtype=jnp.int32)
    for j in range(_C):
        vj = logits[:, j:j + 1]
        ahead = (vj < logits) | ((vj == logits) & (j < col))
        rank = rank + ahead.astype(jnp.int32)
    masked = jnp.where(rank < _K, _FILL, logits)
    rt_ref[...] = jnp.dot(masked, tct_ref[...], preferred_element_type=jnp.float32)


@jax.jit
def _run(x, w1t, b1, w2t, b2, w3t, b3, w4t, b4, tct):
    grid = (_B // _BM,)
    return pl.pallas_call(
        _fused_kernel,
        grid=grid,
        in_specs=[
            pl.BlockSpec((_BM, _F), lambda i: (i, 0)),
            pl.BlockSpec((_F, _H), lambda i: (0, 0)),
            pl.BlockSpec((1, _H), lambda i: (0, 0)),
            pl.BlockSpec((_H, _H), lambda i: (0, 0)),
            pl.BlockSpec((1, _H), lambda i: (0, 0)),
            pl.BlockSpec((_H, _H), lambda i: (0, 0)),
            pl.BlockSpec((1, _H), lambda i: (0, 0)),
            pl.BlockSpec((_H, _C), lambda i: (0, 0)),
            pl.BlockSpec((1, _C), lambda i: (0, 0)),
            pl.BlockSpec((_C, 2), lambda i: (0, 0)),
        ],
        out_specs=[
            pl.BlockSpec((_BM, _C), lambda i: (i, 0)),
            pl.BlockSpec((_BM, 2), lambda i: (i, 0)),
        ],
        out_shape=[
            jax.ShapeDtypeStruct((_B, _C), jnp.float32),
            jax.ShapeDtypeStruct((_B, 2), jnp.float32),
        ],
    )(x, w1t, b1, w2t, b2, w3t, b3, w4t, b4, tct)


def kernel(x_vector, W1, b1, W2, b2, W3, b3, W4, b4, bn_gamma, bn_beta,
           bn_mean, bn_var, topic_course):
    eps = 1e-5
    scale = bn_gamma * jax.lax.rsqrt(bn_var + eps)
    shift = bn_beta - bn_mean * scale
    w1t = (W1 * scale[:, None]).T
    b1f = (b1 * scale + shift)[None, :]
    w2t = (W2 * scale[:, None]).T
    b2f = (b2 * scale + shift)[None, :]
    w3t = (W3 * scale[:, None]).T
    b3f = (b3 * scale + shift)[None, :]
    w4t = W4.T
    b4f = b4[None, :]
    tct = topic_course.T
    logits, rt = _run(x_vector, w1t, b1f, w2t, b2f, w3t, b3f, w4t, b4f, tct)
    return (logits, rt)


# speedup vs baseline: 52.6775x; 6.5159x over previous
import jax
import jax.numpy as jnp
import numpy as np
from jax.experimental import pallas as pl

_B = 16384
_F = 128
_H = 256
_C = 91
_K = 45
_FILL = 0.05
_BM = 4096


def _fused_kernel(x_ref, w1_ref, b1_ref, w2_ref, b2_ref, w3_ref, b3_ref,
                  w4_ref, b4_ref, w4r_ref, b4t_ref, ones_ref, slt_ref,
                  tc_ref, logits_ref, rtt_ref):
    x = x_ref[...]
    h = jnp.maximum(jnp.dot(x, w1_ref[...], preferred_element_type=jnp.float32)
                    + b1_ref[...], 0.0)
    h = jnp.maximum(jnp.dot(h, w2_ref[...], preferred_element_type=jnp.float32)
                    + b2_ref[...], 0.0)
    h = jnp.maximum(jnp.dot(h, w3_ref[...], preferred_element_type=jnp.float32)
                    + b3_ref[...], 0.0)
    logits_ref[...] = jnp.dot(h, w4_ref[...],
                              preferred_element_type=jnp.float32) + b4_ref[...]

    lgt = jax.lax.dot_general(w4r_ref[...], h, (((1,), (1,)), ((), ())),
                              preferred_element_type=jnp.float32) + b4t_ref[...]

    si = jax.lax.bitcast_convert_type(lgt, jnp.int32)
    sm = jnp.where(si < 0, si ^ jnp.int32(0x7FFFFFFF), si)

    ones_row = ones_ref[...]
    p = jnp.full((1, _BM), np.int32(-2**31), dtype=jnp.int32)
    for b in range(31, -1, -1):
        bit = np.int32((1 << b) if b < 31 else -(1 << 31))
        c = p + bit
        cmpf = jnp.where(sm < c, 1.0, 0.0)
        cnt = jnp.dot(ones_row, cmpf, preferred_element_type=jnp.float32)
        p = jnp.where(cnt >= float(_K), p, c)

    lt = sm < p
    ltf = jnp.where(lt, 1.0, 0.0)
    m = jnp.dot(ones_row, ltf, preferred_element_type=jnp.float32)
    eq = sm == p
    eqf = jnp.where(eq, 1.0, 0.0)
    pe = jnp.dot(slt_ref[...], eqf, preferred_element_type=jnp.float32)
    sel = lt | (eq & (pe < (float(_K) - m)))
    maskedt = jnp.where(sel, _FILL, lgt)
    rtt_ref[...] = jnp.dot(tc_ref[...], maskedt,
                           preferred_element_type=jnp.float32)


@jax.jit
def _run(x, w1t, b1, w2t, b2, w3t, b3, w4t, b4, w4r, b4t, ones_row, slt, tc):
    grid = (_B // _BM,)
    return pl.pallas_call(
        _fused_kernel,
        grid=grid,
        in_specs=[
            pl.BlockSpec((_BM, _F), lambda i: (i, 0)),
            pl.BlockSpec((_F, _H), lambda i: (0, 0)),
            pl.BlockSpec((1, _H), lambda i: (0, 0)),
            pl.BlockSpec((_H, _H), lambda i: (0, 0)),
            pl.BlockSpec((1, _H), lambda i: (0, 0)),
            pl.BlockSpec((_H, _H), lambda i: (0, 0)),
            pl.BlockSpec((1, _H), lambda i: (0, 0)),
            pl.BlockSpec((_H, _C), lambda i: (0, 0)),
            pl.BlockSpec((1, _C), lambda i: (0, 0)),
            pl.BlockSpec((_C, _H), lambda i: (0, 0)),
            pl.BlockSpec((_C, 1), lambda i: (0, 0)),
            pl.BlockSpec((1, _C), lambda i: (0, 0)),
            pl.BlockSpec((_C, _C), lambda i: (0, 0)),
            pl.BlockSpec((2, _C), lambda i: (0, 0)),
        ],
        out_specs=[
            pl.BlockSpec((_BM, _C), lambda i: (i, 0)),
            pl.BlockSpec((2, _BM), lambda i: (0, i)),
        ],
        out_shape=[
            jax.ShapeDtypeStruct((_B, _C), jnp.float32),
            jax.ShapeDtypeStruct((2, _B), jnp.float32),
        ],
    )(x, w1t, b1, w2t, b2, w3t, b3, w4t, b4, w4r, b4t, ones_row, slt, tc)


def kernel(x_vector, W1, b1, W2, b2, W3, b3, W4, b4, bn_gamma, bn_beta,
           bn_mean, bn_var, topic_course):
    eps = 1e-5
    scale = bn_gamma * jax.lax.rsqrt(bn_var + eps)
    shift = bn_beta - bn_mean * scale
    w1t = (W1 * scale[:, None]).T
    b1f = (b1 * scale + shift)[None, :]
    w2t = (W2 * scale[:, None]).T
    b2f = (b2 * scale + shift)[None, :]
    w3t = (W3 * scale[:, None]).T
    b3f = (b3 * scale + shift)[None, :]
    w4t = W4.T
    b4f = b4[None, :]
    b4t = b4[:, None]
    ones_row = jnp.ones((1, _C), dtype=jnp.float32)
    slt = jnp.asarray(np.tril(np.ones((_C, _C), dtype=np.float32), k=-1))
    logits, rtt = _run(x_vector, w1t, b1f, w2t, b2f, w3t, b3f, w4t, b4f,
                       W4, b4t, ones_row, slt, topic_course)
    return (logits, rtt.T)
